# baseline (speedup 1.0000x reference)
import functools

import jax
import jax.numpy as jnp
from jax import lax
from jax.experimental import pallas as pl
from jax.experimental.pallas import tpu as pltpu
from jax.experimental.pallas import tpu_sc as plsc

N = 10000
E = 160000
Z = 4
F = 128
NB = 8
R_MAX = 5.0
P = 5.0
NG = 1
AVG_NEIGH = 16.0
NLAYERS = 2

NC = 2
NS = 16
NW = NC * NS
E_PAD = 163840
EPT = E_PAD // NW


def _bessel(r):
    n = jnp.arange(1, NB + 1, dtype=jnp.float32)
    rr = jnp.clip(r, 1e-6, None)
    return jnp.sqrt(2.0 / R_MAX) * jnp.sin(n[None, :] * jnp.pi * rr[:, None] / R_MAX) / rr[:, None]


def _poly_cutoff(r):
    x = r / R_MAX
    p = P
    env = 1.0 - ((p + 1.0) * (p + 2.0) / 2.0) * x ** p + p * (p + 2.0) * x ** (p + 1.0) - (p * (p + 1.0) / 2.0) * x ** (p + 2.0)
    return jnp.where(x < 1.0, env, 0.0)


def _sph(vec):
    nrm = jnp.clip(jnp.linalg.norm(vec, axis=-1, keepdims=True), 1e-9, None)
    u = vec / nrm
    s3 = jnp.sqrt(3.0)
    return jnp.stack([jnp.ones_like(u[:, 0]), s3 * u[:, 1], s3 * u[:, 2], s3 * u[:, 0]], axis=-1)


def _s1_body(px, py, pz, es, er, dx, dy, dz,
             pxv, pyv, pzv, esv, erv, dxv, dyv, dzv):
    wid = lax.axis_index("s") * NC + lax.axis_index("c")
    base = wid * EPT
    pltpu.sync_copy(px, pxv)
    pltpu.sync_copy(py, pyv)
    pltpu.sync_copy(pz, pzv)
    pltpu.sync_copy(es.at[pl.ds(base, EPT)], esv)
    pltpu.sync_copy(er.at[pl.ds(base, EPT)], erv)

    def step(i, carry):
        o = i * 16
        s_idx = esv[pl.ds(o, 16)]
        r_idx = erv[pl.ds(o, 16)]
        dxv[pl.ds(o, 16)] = (plsc.load_gather(pxv, [r_idx])
                             - plsc.load_gather(pxv, [s_idx]))
        dyv[pl.ds(o, 16)] = (plsc.load_gather(pyv, [r_idx])
                             - plsc.load_gather(pyv, [s_idx]))
        dzv[pl.ds(o, 16)] = (plsc.load_gather(pzv, [r_idx])
                             - plsc.load_gather(pzv, [s_idx]))
        return carry

    lax.fori_loop(0, EPT // 16, step, 0)
    pltpu.sync_copy(dxv, dx.at[pl.ds(base, EPT)])
    pltpu.sync_copy(dyv, dy.at[pl.ds(base, EPT)])
    pltpu.sync_copy(dzv, dz.at[pl.ds(base, EPT)])


def _edge_vectors(px, py, pz, es, er):
    mesh = plsc.VectorSubcoreMesh(core_axis_name="c", subcore_axis_name="s")
    f32 = jnp.float32
    out = jax.ShapeDtypeStruct((E_PAD,), f32)
    k = pl.kernel(
        _s1_body,
        out_type=(out, out, out),
        mesh=mesh,
        compiler_params=pltpu.CompilerParams(needs_layout_passes=False),
        scratch_types=(
            pltpu.VMEM((N,), f32),
            pltpu.VMEM((N,), f32),
            pltpu.VMEM((N,), f32),
            pltpu.VMEM((EPT,), jnp.int32),
            pltpu.VMEM((EPT,), jnp.int32),
            pltpu.VMEM((EPT,), f32),
            pltpu.VMEM((EPT,), f32),
            pltpu.VMEM((EPT,), f32),
        ),
    )
    return k(px, py, pz, es, er)


N_ACC = 10112
EPT2 = E_PAD // NS
WINS = EPT2 // 128
RPT = N_ACC // NS


WN = 64
WINS = EPT2 // WN


def _s2_body(pre, tpw, shb, es, ersc, zrows, out,
             iv0, iv1, ri0, ri1, sb0, sb1, tv0, tv1, gv0, gv1, acc,
             si0, si1, st0, st1, sg0, sg1, ss0, ss1):
    c = lax.axis_index("c")
    s = lax.axis_index("s")
    ebase = s * EPT2
    rbase = s * RPT
    IV = (iv0, iv1)
    RI = (ri0, ri1)
    SB = (sb0, sb1)
    TV = (tv0, tv1)
    GV = (gv0, gv1)
    SI = (si0, si1)
    ST = (st0, st1)
    SG = (sg0, sg1)
    SS = (ss0, ss1)

    for p in range(2):
        ch = 2 * p + c

        def idx_issue(w, b):
            pltpu.async_copy(es.at[pl.ds(ebase + w * WN, WN)], IV[b], SI[b])

        def meta_issue(w, b):
            e0 = ebase + w * WN
            pltpu.async_copy(tpw.at[pl.ds(e0, WN)], TV[b], ST[b])
            pltpu.async_copy(shb.at[ch, pl.ds(e0, WN)], SB[b], ST[b])
            pltpu.async_copy(ersc.at[pl.ds(e0, WN)], RI[b], ST[b])

        def meta_wait(b):
            pltpu.make_async_copy(tpw.at[pl.ds(0, WN)], TV[b], ST[b]).wait()
            pltpu.make_async_copy(shb.at[0, pl.ds(0, WN)], SB[b], ST[b]).wait()
            pltpu.make_async_copy(ersc.at[pl.ds(0, WN)], RI[b], ST[b]).wait()

        pltpu.sync_copy(zrows, acc.at[pl.ds(rbase, RPT)])
        plsc.subcore_barrier()

        idx_issue(0, 0)
        idx_issue(1, 1)
        meta_issue(0, 0)
        pltpu.make_async_copy(es.at[pl.ds(0, WN)], IV[0], SI[0]).wait()
        pltpu.async_copy(pre.at[IV[0]], GV[0], SG[0])

        def window(w, b):
            nb = 1 - b

            @pl.when(w + 1 < WINS)
            def _():
                pltpu.make_async_copy(es.at[pl.ds(0, WN)], IV[nb],
                                      SI[nb]).wait()
                pltpu.async_copy(pre.at[IV[nb]], GV[nb], SG[nb])

            @pl.when(w >= 1)
            def _():
                pltpu.make_async_copy(TV[nb], acc.at[RI[nb]], SS[nb]).wait()

            @pl.when(w + 1 < WINS)
            def _():
                meta_issue(w + 1, nb)

            meta_wait(b)
            pltpu.make_async_copy(pre.at[IV[b]], GV[b], SG[b]).wait()

            @pl.when(w + 2 < WINS)
            def _():
                idx_issue(w + 2, b)

            sb = SB[b]
            tv = TV[b]
            gv = GV[b]

            def edge4(e4, cc):
                for u in range(4):
                    e = e4 * 4 + u
                    sv = sb[e, pl.ds(0, 16)]
                    for k in range(8):
                        sl = pl.ds(k * 16, 16)
                        tv[e, sl] = gv[e, sl] * tv[e, sl] * sv
                return cc

            lax.fori_loop(0, WN // 4, edge4, 0)
            pltpu.async_copy(TV[b], acc.at[RI[b]], SS[b], add=True)

        def pair(i, carry):
            window(2 * i, 0)
            window(2 * i + 1, 1)
            return carry

        lax.fori_loop(0, WINS // 2, pair, 0)
        pltpu.make_async_copy(TV[1], acc.at[RI[1]], SS[1]).wait()
        plsc.subcore_barrier()
        pltpu.sync_copy(acc.at[pl.ds(rbase, RPT)],
                        out.at[ch, pl.ds(rbase, RPT)])
        plsc.subcore_barrier()


def _aggregate(pre, tpw, shb, es, ersc, zrows):
    mesh = plsc.VectorSubcoreMesh(core_axis_name="c", subcore_axis_name="s")
    f32 = jnp.float32
    i32 = jnp.int32
    k = pl.kernel(
        _s2_body,
        out_type=jax.ShapeDtypeStruct((4, N_ACC, F), f32),
        mesh=mesh,
        compiler_params=pltpu.CompilerParams(needs_layout_passes=False),
        scratch_types=(
            pltpu.VMEM((WN,), i32), pltpu.VMEM((WN,), i32),
            pltpu.VMEM((WN,), i32), pltpu.VMEM((WN,), i32),
            pltpu.VMEM((WN, 16), f32), pltpu.VMEM((WN, 16), f32),
            pltpu.VMEM((WN, F), f32), pltpu.VMEM((WN, F), f32),
            pltpu.VMEM((WN, F), f32), pltpu.VMEM((WN, F), f32),
            pltpu.VMEM_SHARED((N_ACC, F), f32),
            pltpu.SemaphoreType.DMA, pltpu.SemaphoreType.DMA,
            pltpu.SemaphoreType.DMA, pltpu.SemaphoreType.DMA,
            pltpu.SemaphoreType.DMA, pltpu.SemaphoreType.DMA,
            pltpu.SemaphoreType.DMA, pltpu.SemaphoreType.DMA,
        ),
    )
    return k(pre, tpw, shb, es, ersc, zrows)


EB = 2048
ER = E_PAD // 128
EBR = EB // 128


def _t1a_body(dx, dy, dz, ef8, sh3):
    x1 = dx[...]
    x2 = dy[...]
    x3 = dz[...]
    r2 = x1 * x1 + x2 * x2 + x3 * x3
    r = jnp.sqrt(r2)
    inv = 1.0 / jnp.maximum(r, 1e-9)
    s3 = jnp.sqrt(jnp.float32(3.0))
    sh3[...] = jnp.stack([s3 * x2 * inv, s3 * x3 * inv, s3 * x1 * inv], axis=0)
    rr = jnp.maximum(r, 1e-6)
    xc = r * (1.0 / R_MAX)
    p = P
    x2c = xc * xc
    x4c = x2c * x2c
    x5c = x4c * xc
    env = (1.0 - ((p + 1.0) * (p + 2.0) / 2.0) * x5c + p * (p + 2.0) * x5c * xc
           - (p * (p + 1.0) / 2.0) * x5c * x2c)
    cut = jnp.where(xc < 1.0, env, 0.0)
    pref = jnp.sqrt(2.0 / R_MAX) * cut / rr
    theta = (jnp.pi / R_MAX) * rr
    ef8[...] = jnp.stack([pref * jnp.sin((i + 1.0) * theta) for i in range(NB)],
                         axis=0)


def _t1a(dx2, dy2, dz2):
    f32 = jnp.float32
    blk = lambda: pl.BlockSpec((EBR, 128), lambda i: (i, 0))
    return pl.pallas_call(
        _t1a_body,
        grid=(ER // EBR,),
        in_specs=[blk(), blk(), blk()],
        out_specs=[pl.BlockSpec((NB, EBR, 128), lambda i: (0, i, 0)),
                   pl.BlockSpec((3, EBR, 128), lambda i: (0, i, 0))],
        out_shape=[jax.ShapeDtypeStruct((NB, ER, 128), f32),
                   jax.ShapeDtypeStruct((3, ER, 128), f32)],
    )(dx2, dy2, dz2)


def _silu_t(v):
    return 0.5 * v * (1.0 + jnp.tanh(0.5 * v))


def _t1b_body(ef, w1t0, w2t0, w3t0, w40, w1t1, w2t1, w3t1, w41, tpw0, tpw1):
    f32 = jnp.float32
    cdim = (((1,), (0,)), ((), ()))
    e = ef[...]
    for (wa, wb, wc, wd, out) in ((w1t0, w2t0, w3t0, w40, tpw0),
                                  (w1t1, w2t1, w3t1, w41, tpw1)):
        h = _silu_t(lax.dot_general(wa[...], e, cdim,
                                    preferred_element_type=f32))
        h = _silu_t(lax.dot_general(wb[...], h, cdim,
                                    preferred_element_type=f32))
        h = _silu_t(lax.dot_general(wc[...], h, cdim,
                                    preferred_element_type=f32))
        out[...] = lax.dot_general(h, wd[...], (((0,), (0,)), ((), ())),
                                   preferred_element_type=f32)


def _t1b(ef2, W1, W2, W3, W4):
    f32 = jnp.float32
    wspec = lambda a, b: pl.BlockSpec((a, b), lambda i: (0, 0))
    return pl.pallas_call(
        _t1b_body,
        grid=(E_PAD // EB,),
        in_specs=[pl.BlockSpec((NB, EB), lambda i: (0, i)),
                  wspec(64, NB), wspec(64, 64), wspec(64, 64), wspec(64, F),
                  wspec(64, NB), wspec(64, 64), wspec(64, 64), wspec(64, F)],
        out_specs=[pl.BlockSpec((EB, F), lambda i: (i, 0)),
                   pl.BlockSpec((EB, F), lambda i: (i, 0))],
        out_shape=[jax.ShapeDtypeStruct((E_PAD, F), f32),
                   jax.ShapeDtypeStruct((E_PAD, F), f32)],
    )(ef2, W1[0].T, W2[0].T, W3[0].T, W4[0],
      W1[1].T, W2[1].T, W3[1].T, W4[1])


def _t0_body(na, wem, pre):
    pre[...] = jnp.dot(na[...], wem[...], preferred_element_type=jnp.float32)


def _t0(node_attrs, wem):
    return pl.pallas_call(
        _t0_body,
        grid=(5,),
        in_specs=[pl.BlockSpec((N // 5, Z), lambda i: (i, 0)),
                  pl.BlockSpec((Z, F), lambda i: (0, 0))],
        out_specs=pl.BlockSpec((N // 5, F), lambda i: (i, 0)),
        out_shape=jax.ShapeDtypeStruct((N, F), jnp.float32),
    )(node_attrs, wem)


NBLK = N_ACC // 8


def _t2_body(acc4, wprod, wread, wq, wmsg, pre_next, er, qv, has_next):
    i = pl.program_id(0)
    a0 = acc4[0] * (1.0 / AVG_NEIGH)
    a1 = acc4[1] * (1.0 / AVG_NEIGH)
    a2 = acc4[2] * (1.0 / AVG_NEIGH)
    a3 = acc4[3] * (1.0 / AVG_NEIGH)
    x = a0 + a1 * a1 + a2 * a2 + a3 * a3
    f32 = jnp.float32
    feats = (x * jax.nn.sigmoid(x)) @ wprod[...]
    rowid = i * NBLK + jax.lax.broadcasted_iota(jnp.int32, (NBLK, 1), 0)
    valid = rowid < N
    erp = jnp.sum(jnp.where(valid, jnp.dot(feats, wread[...],
                                           preferred_element_type=f32), 0.0))

    @pl.when(i == 0)
    def _():
        er[...] = jnp.zeros_like(er)

    er[...] = er[...] + erp
    qv[...] = jnp.dot(feats, wq[...], preferred_element_type=f32)
    if has_next:
        pre_next[...] = jnp.dot(feats, wmsg[...], preferred_element_type=f32)


def _t2(acc4, wprod, wread, wq, wmsg, has_next):
    f32 = jnp.float32
    body = functools.partial(_t2_body, has_next=has_next)
    if not has_next:
        def body(acc4, wprod, wread, wq, wmsg, er, qv):
            return _t2_body(acc4, wprod, wread, wq, wmsg, None, er, qv, False)
    outs = [jax.ShapeDtypeStruct((N_ACC, F), f32),
            jax.ShapeDtypeStruct((1, 1), f32),
            jax.ShapeDtypeStruct((N_ACC, 1), f32)]
    out_specs = [pl.BlockSpec((NBLK, F), lambda i: (i, 0)),
                 pl.BlockSpec((1, 1), lambda i: (0, 0)),
                 pl.BlockSpec((NBLK, 1), lambda i: (i, 0))]
    if not has_next:
        outs = outs[1:]
        out_specs = out_specs[1:]
    res = pl.pallas_call(
        body,
        grid=(8,),
        in_specs=[pl.BlockSpec((4, NBLK, F), lambda i: (0, i, 0)),
                  pl.BlockSpec((F, F), lambda i: (0, 0)),
                  pl.BlockSpec((F, 1), lambda i: (0, 0)),
                  pl.BlockSpec((F, 1), lambda i: (0, 0)),
                  pl.BlockSpec((F, F), lambda i: (0, 0))],
        out_specs=out_specs,
        out_shape=outs,
    )(acc4, wprod, wread, wq, wmsg)
    if not has_next:
        return (None,) + tuple(res)
    return res


def _t3_body(na, we0, ch, q0, q1, pos, er0, er1, out, accv):
    i = pl.program_id(0)

    @pl.when(i == 0)
    def _():
        accv[...] = jnp.zeros_like(accv)

    f32 = jnp.float32
    e0p = jnp.sum(jnp.dot(na[...], we0[...], preferred_element_type=f32))
    tq = ch[...] + q0[...] + q1[...]
    ecp = jnp.sum(tq * tq)
    dip = jnp.sum(pos[...] * tq, axis=0, keepdims=True)
    lane = jax.lax.broadcasted_iota(jnp.int32, (1, 128), 1)
    row = (jnp.where(lane == 0, e0p, 0.0) + jnp.where(lane == 1, ecp, 0.0)
           + jnp.where(lane == 2, dip[0, 0], 0.0)
           + jnp.where(lane == 3, dip[0, 1], 0.0)
           + jnp.where(lane == 4, dip[0, 2], 0.0))
    accv[...] += row

    @pl.when(i == pl.num_programs(0) - 1)
    def _():
        a = accv[...]
        dip2 = a[0, 2] ** 2 + a[0, 3] ** 2 + a[0, 4] ** 2
        val = (er0[0, 0] + er1[0, 0] + a[0, 0] + 0.5 * a[0, 1] + 1e-6 * dip2)
        out[...] = jnp.full((1, 1), val, jnp.float32)


def _t3(node_attrs, we0, charges, q0, q1, positions, er0, er1):
    f32 = jnp.float32
    nb = N // 5
    return pl.pallas_call(
        _t3_body,
        grid=(5,),
        in_specs=[pl.BlockSpec((nb, Z), lambda i: (i, 0)),
                  pl.BlockSpec((Z, 1), lambda i: (0, 0)),
                  pl.BlockSpec((nb, 1), lambda i: (i, 0)),
                  pl.BlockSpec((nb, 1), lambda i: (i, 0)),
                  pl.BlockSpec((nb, 1), lambda i: (i, 0)),
                  pl.BlockSpec((nb, 3), lambda i: (i, 0)),
                  pl.BlockSpec((1, 1), lambda i: (0, 0)),
                  pl.BlockSpec((1, 1), lambda i: (0, 0))],
        out_specs=pl.BlockSpec((1, 1), lambda i: (0, 0)),
        out_shape=jax.ShapeDtypeStruct((1, 1), f32),
        scratch_shapes=[pltpu.VMEM((1, 128), f32)],
    )(node_attrs, we0, charges, q0, q1, positions, er0, er1)


def kernel(positions, node_attrs, charges, shifts, W_E0, W_embed, W_msg, W1, W2, W3, W4, W_prod, W_read, W_q, edge_index, batch, ptr):
    n_nodes = positions.shape[0]
    sender = edge_index[0]
    recv = edge_index[1]

    px = positions[:, 0]
    py = positions[:, 1]
    pz = positions[:, 2]
    padidx = (jnp.arange(E, E_PAD, dtype=jnp.int32) % N).astype(jnp.int32)
    es_pad = jnp.concatenate([sender.astype(jnp.int32), padidx])
    er_pad = jnp.concatenate([recv.astype(jnp.int32), padidx])
    ersc = jnp.concatenate([
        recv.astype(jnp.int32),
        N + (jnp.arange(E, E_PAD, dtype=jnp.int32) % 16),
    ])
    dx, dy, dz = _edge_vectors(px, py, pz, es_pad, er_pad)

    ef8, sh3 = _t1a(dx.reshape(ER, 128), dy.reshape(ER, 128),
                    dz.reshape(ER, 128))
    ef2 = ef8.reshape(NB, E_PAD)
    tpw0, tpw1 = _t1b(ef2, W1, W2, W3, W4)
    sh_s = sh3.reshape(3, E_PAD)
    sh_b = jnp.concatenate(
        [jnp.ones((1, E_PAD), jnp.float32), sh_s], axis=0)
    sh_b = jnp.broadcast_to(sh_b[:, :, None], (4, E_PAD, 16))
    zrows = jnp.zeros((RPT, F), jnp.float32)

    pre = _t0(node_attrs, W_embed @ W_msg[0])

    acc4 = _aggregate(pre, tpw0, sh_b, es_pad, ersc, zrows)
    pre1, er0, q0 = _t2(acc4, W_prod[0], W_read[0][:, None], W_q[0][:, None],
                        W_msg[1], True)
    acc4 = _aggregate(pre1, tpw1, sh_b, es_pad, ersc, zrows)
    _, er1, q1 = _t2(acc4, W_prod[1], W_read[1][:, None], W_q[1][:, None],
                     W_msg[1], False)

    out = _t3(node_attrs, W_E0[:, None], charges[:, None],
              q0[:N], q1[:N], positions, er0, er1)
    return out[0]


# speedup vs baseline: 1.0265x; 1.0265x over previous
import functools

import jax
import jax.numpy as jnp
from jax import lax
from jax.experimental import pallas as pl
from jax.experimental.pallas import tpu as pltpu
from jax.experimental.pallas import tpu_sc as plsc

N = 10000
E = 160000
Z = 4
F = 128
NB = 8
R_MAX = 5.0
P = 5.0
NG = 1
AVG_NEIGH = 16.0
NLAYERS = 2

NC = 2
NS = 16
NW = NC * NS
E_PAD = 163840
EPT = E_PAD // NW


def _s1_body(px, py, pz, es, er, dx, dy, dz,
             pxv, pyv, pzv, esv, erv, dxv, dyv, dzv):
    wid = lax.axis_index("s") * NC + lax.axis_index("c")
    base = wid * EPT
    pltpu.sync_copy(px, pxv)
    pltpu.sync_copy(py, pyv)
    pltpu.sync_copy(pz, pzv)
    pltpu.sync_copy(es.at[pl.ds(base, EPT)], esv)
    pltpu.sync_copy(er.at[pl.ds(base, EPT)], erv)

    def step(i, carry):
        o = i * 16
        s_idx = esv[pl.ds(o, 16)]
        r_idx = erv[pl.ds(o, 16)]
        dxv[pl.ds(o, 16)] = (plsc.load_gather(pxv, [r_idx])
                             - plsc.load_gather(pxv, [s_idx]))
        dyv[pl.ds(o, 16)] = (plsc.load_gather(pyv, [r_idx])
                             - plsc.load_gather(pyv, [s_idx]))
        dzv[pl.ds(o, 16)] = (plsc.load_gather(pzv, [r_idx])
                             - plsc.load_gather(pzv, [s_idx]))
        return carry

    lax.fori_loop(0, EPT // 16, step, 0)
    pltpu.sync_copy(dxv, dx.at[pl.ds(base, EPT)])
    pltpu.sync_copy(dyv, dy.at[pl.ds(base, EPT)])
    pltpu.sync_copy(dzv, dz.at[pl.ds(base, EPT)])


def _edge_vectors(px, py, pz, es, er):
    mesh = plsc.VectorSubcoreMesh(core_axis_name="c", subcore_axis_name="s")
    f32 = jnp.float32
    out = jax.ShapeDtypeStruct((E_PAD,), f32)
    k = pl.kernel(
        _s1_body,
        out_type=(out, out, out),
        mesh=mesh,
        compiler_params=pltpu.CompilerParams(needs_layout_passes=False),
        scratch_types=(
            pltpu.VMEM((N,), f32),
            pltpu.VMEM((N,), f32),
            pltpu.VMEM((N,), f32),
            pltpu.VMEM((EPT,), jnp.int32),
            pltpu.VMEM((EPT,), jnp.int32),
            pltpu.VMEM((EPT,), f32),
            pltpu.VMEM((EPT,), f32),
            pltpu.VMEM((EPT,), f32),
        ),
    )
    return k(px, py, pz, es, er)


N_ACC = 10112
EPT2 = E_PAD // NS
WINS = EPT2 // 128
RPT = N_ACC // NS


WN = 64
WINS = EPT2 // WN


def _s2_body(pre, tpw, shb, es, ersc, zrows, out,
             iv0, iv1, ri0, ri1, sb0, sb1, tv0, tv1, gv0, gv1, acc,
             si0, si1, st0, st1, sg0, sg1, ss0, ss1):
    c = lax.axis_index("c")
    s = lax.axis_index("s")
    ebase = s * EPT2
    rbase = s * RPT
    IV = (iv0, iv1)
    RI = (ri0, ri1)
    SB = (sb0, sb1)
    TV = (tv0, tv1)
    GV = (gv0, gv1)
    SI = (si0, si1)
    ST = (st0, st1)
    SG = (sg0, sg1)
    SS = (ss0, ss1)

    for p in range(2):
        ch = 2 * p + c

        def idx_issue(w, b):
            pltpu.async_copy(es.at[pl.ds(ebase + w * WN, WN)], IV[b], SI[b])

        def meta_issue(w, b):
            e0 = ebase + w * WN
            pltpu.async_copy(tpw.at[pl.ds(e0, WN)], TV[b], ST[b])
            pltpu.async_copy(shb.at[ch, pl.ds(e0, WN)], SB[b], ST[b])
            pltpu.async_copy(ersc.at[pl.ds(e0, WN)], RI[b], ST[b])

        def meta_wait(b):
            pltpu.make_async_copy(tpw.at[pl.ds(0, WN)], TV[b], ST[b]).wait()
            pltpu.make_async_copy(shb.at[0, pl.ds(0, WN)], SB[b], ST[b]).wait()
            pltpu.make_async_copy(ersc.at[pl.ds(0, WN)], RI[b], ST[b]).wait()

        pltpu.sync_copy(zrows, acc.at[pl.ds(rbase, RPT)])
        plsc.subcore_barrier()

        idx_issue(0, 0)
        idx_issue(1, 1)
        meta_issue(0, 0)
        pltpu.make_async_copy(es.at[pl.ds(0, WN)], IV[0], SI[0]).wait()
        pltpu.async_copy(pre.at[IV[0]], GV[0], SG[0])

        def window(w, b):
            nb = 1 - b

            @pl.when(w + 1 < WINS)
            def _():
                pltpu.make_async_copy(es.at[pl.ds(0, WN)], IV[nb],
                                      SI[nb]).wait()
                pltpu.async_copy(pre.at[IV[nb]], GV[nb], SG[nb])

            @pl.when(w >= 1)
            def _():
                pltpu.make_async_copy(TV[nb], acc.at[RI[nb]], SS[nb]).wait()

            @pl.when(w + 1 < WINS)
            def _():
                meta_issue(w + 1, nb)

            meta_wait(b)
            pltpu.make_async_copy(pre.at[IV[b]], GV[b], SG[b]).wait()

            @pl.when(w + 2 < WINS)
            def _():
                idx_issue(w + 2, b)

            sb = SB[b]
            tv = TV[b]
            gv = GV[b]

            def edge4(e4, cc):
                for u in range(4):
                    e = e4 * 4 + u
                    sv = sb[e, pl.ds(0, 16)]
                    for k in range(8):
                        sl = pl.ds(k * 16, 16)
                        tv[e, sl] = gv[e, sl] * tv[e, sl] * sv
                return cc

            lax.fori_loop(0, WN // 4, edge4, 0)
            pltpu.async_copy(TV[b], acc.at[RI[b]], SS[b], add=True)

        def pair(i, carry):
            window(2 * i, 0)
            window(2 * i + 1, 1)
            return carry

        lax.fori_loop(0, WINS // 2, pair, 0)
        pltpu.make_async_copy(TV[1], acc.at[RI[1]], SS[1]).wait()
        plsc.subcore_barrier()
        pltpu.sync_copy(acc.at[pl.ds(rbase, RPT)],
                        out.at[ch, pl.ds(rbase, RPT)])
        plsc.subcore_barrier()


def _aggregate(pre, tpw, shb, es, ersc, zrows):
    mesh = plsc.VectorSubcoreMesh(core_axis_name="c", subcore_axis_name="s")
    f32 = jnp.float32
    i32 = jnp.int32
    k = pl.kernel(
        _s2_body,
        out_type=jax.ShapeDtypeStruct((4, N_ACC, F), f32),
        mesh=mesh,
        compiler_params=pltpu.CompilerParams(needs_layout_passes=False),
        scratch_types=(
            pltpu.VMEM((WN,), i32), pltpu.VMEM((WN,), i32),
            pltpu.VMEM((WN,), i32), pltpu.VMEM((WN,), i32),
            pltpu.VMEM((WN, 16), f32), pltpu.VMEM((WN, 16), f32),
            pltpu.VMEM((WN, F), f32), pltpu.VMEM((WN, F), f32),
            pltpu.VMEM((WN, F), f32), pltpu.VMEM((WN, F), f32),
            pltpu.VMEM_SHARED((N_ACC, F), f32),
            pltpu.SemaphoreType.DMA, pltpu.SemaphoreType.DMA,
            pltpu.SemaphoreType.DMA, pltpu.SemaphoreType.DMA,
            pltpu.SemaphoreType.DMA, pltpu.SemaphoreType.DMA,
            pltpu.SemaphoreType.DMA, pltpu.SemaphoreType.DMA,
        ),
    )
    return k(pre, tpw, shb, es, ersc, zrows)


EB = 2048
ER = E_PAD // 128
EBR = EB // 128


def _t1a_body(dx, dy, dz, ef8, sh3):
    x1 = dx[...]
    x2 = dy[...]
    x3 = dz[...]
    r2 = x1 * x1 + x2 * x2 + x3 * x3
    r = jnp.sqrt(r2)
    inv = 1.0 / jnp.maximum(r, 1e-9)
    s3 = jnp.sqrt(jnp.float32(3.0))
    sh3[...] = jnp.stack([s3 * x2 * inv, s3 * x3 * inv, s3 * x1 * inv], axis=0)
    rr = jnp.maximum(r, 1e-6)
    xc = r * (1.0 / R_MAX)
    p = P
    x2c = xc * xc
    x4c = x2c * x2c
    x5c = x4c * xc
    env = (1.0 - ((p + 1.0) * (p + 2.0) / 2.0) * x5c + p * (p + 2.0) * x5c * xc
           - (p * (p + 1.0) / 2.0) * x5c * x2c)
    cut = jnp.where(xc < 1.0, env, 0.0)
    pref = jnp.sqrt(2.0 / R_MAX) * cut / rr
    theta = (jnp.pi / R_MAX) * rr
    ef8[...] = jnp.stack([pref * jnp.sin((i + 1.0) * theta) for i in range(NB)],
                         axis=0)


def _t1a(dx2, dy2, dz2):
    f32 = jnp.float32
    blk = lambda: pl.BlockSpec((EBR, 128), lambda i: (i, 0))
    return pl.pallas_call(
        _t1a_body,
        grid=(ER // EBR,),
        in_specs=[blk(), blk(), blk()],
        out_specs=[pl.BlockSpec((NB, EBR, 128), lambda i: (0, i, 0)),
                   pl.BlockSpec((3, EBR, 128), lambda i: (0, i, 0))],
        out_shape=[jax.ShapeDtypeStruct((NB, ER, 128), f32),
                   jax.ShapeDtypeStruct((3, ER, 128), f32)],
    )(dx2, dy2, dz2)


def _silu_t(v):
    return 0.5 * v * (1.0 + jnp.tanh(0.5 * v))


def _t1b_body(ef, w1t, w2t, w3t, w4, tpw):
    f32 = jnp.float32
    cdim = (((1,), (0,)), ((), ()))
    h = _silu_t(lax.dot_general(w1t[...], ef[...], cdim,
                                preferred_element_type=f32))
    h = _silu_t(lax.dot_general(w2t[...], h, cdim,
                                preferred_element_type=f32))
    h = _silu_t(lax.dot_general(w3t[...], h, cdim,
                                preferred_element_type=f32))
    tpw[...] = lax.dot_general(h, w4[...], (((0,), (0,)), ((), ())),
                               preferred_element_type=f32)


def _t1b(ef2, w1t, w2t, w3t, w4):
    f32 = jnp.float32
    wspec = lambda a, b: pl.BlockSpec((a, b), lambda i: (0, 0))
    return pl.pallas_call(
        _t1b_body,
        grid=(E_PAD // EB,),
        in_specs=[pl.BlockSpec((NB, EB), lambda i: (0, i)),
                  wspec(64, NB), wspec(64, 64), wspec(64, 64), wspec(64, F)],
        out_specs=pl.BlockSpec((EB, F), lambda i: (i, 0)),
        out_shape=jax.ShapeDtypeStruct((E_PAD, F), f32),
    )(ef2, w1t, w2t, w3t, w4)


def _t0_body(na, wem, pre):
    pre[...] = jnp.dot(na[...], wem[...], preferred_element_type=jnp.float32)


def _t0(node_attrs, wem):
    return pl.pallas_call(
        _t0_body,
        grid=(5,),
        in_specs=[pl.BlockSpec((N // 5, Z), lambda i: (i, 0)),
                  pl.BlockSpec((Z, F), lambda i: (0, 0))],
        out_specs=pl.BlockSpec((N // 5, F), lambda i: (i, 0)),
        out_shape=jax.ShapeDtypeStruct((N, F), jnp.float32),
    )(node_attrs, wem)


NBLK = N_ACC // 8


def _t2_body(acc4, wprod, wread, wq, wmsg, pre_next, er, qv, has_next):
    i = pl.program_id(0)
    a0 = acc4[0] * (1.0 / AVG_NEIGH)
    a1 = acc4[1] * (1.0 / AVG_NEIGH)
    a2 = acc4[2] * (1.0 / AVG_NEIGH)
    a3 = acc4[3] * (1.0 / AVG_NEIGH)
    x = a0 + a1 * a1 + a2 * a2 + a3 * a3
    f32 = jnp.float32
    feats = (x * jax.nn.sigmoid(x)) @ wprod[...]
    rowid = i * NBLK + jax.lax.broadcasted_iota(jnp.int32, (NBLK, 1), 0)
    valid = rowid < N
    erp = jnp.sum(jnp.where(valid, jnp.dot(feats, wread[...],
                                           preferred_element_type=f32), 0.0))

    @pl.when(i == 0)
    def _():
        er[...] = jnp.zeros_like(er)

    er[...] = er[...] + erp
    qv[...] = jnp.dot(feats, wq[...], preferred_element_type=f32)
    if has_next:
        pre_next[...] = jnp.dot(feats, wmsg[...], preferred_element_type=f32)


def _t2(acc4, wprod, wread, wq, wmsg, has_next):
    f32 = jnp.float32
    body = functools.partial(_t2_body, has_next=has_next)
    if not has_next:
        def body(acc4, wprod, wread, wq, wmsg, er, qv):
            return _t2_body(acc4, wprod, wread, wq, wmsg, None, er, qv, False)
    outs = [jax.ShapeDtypeStruct((N_ACC, F), f32),
            jax.ShapeDtypeStruct((1, 1), f32),
            jax.ShapeDtypeStruct((N_ACC, 1), f32)]
    out_specs = [pl.BlockSpec((NBLK, F), lambda i: (i, 0)),
                 pl.BlockSpec((1, 1), lambda i: (0, 0)),
                 pl.BlockSpec((NBLK, 1), lambda i: (i, 0))]
    if not has_next:
        outs = outs[1:]
        out_specs = out_specs[1:]
    res = pl.pallas_call(
        body,
        grid=(8,),
        in_specs=[pl.BlockSpec((4, NBLK, F), lambda i: (0, i, 0)),
                  pl.BlockSpec((F, F), lambda i: (0, 0)),
                  pl.BlockSpec((F, 1), lambda i: (0, 0)),
                  pl.BlockSpec((F, 1), lambda i: (0, 0)),
                  pl.BlockSpec((F, F), lambda i: (0, 0))],
        out_specs=out_specs,
        out_shape=outs,
    )(acc4, wprod, wread, wq, wmsg)
    if not has_next:
        return (None,) + tuple(res)
    return res


def _t3_body(na, we0, ch, q0, q1, pos, er0, er1, out, accv):
    i = pl.program_id(0)

    @pl.when(i == 0)
    def _():
        accv[...] = jnp.zeros_like(accv)

    f32 = jnp.float32
    e0p = jnp.sum(jnp.dot(na[...], we0[...], preferred_element_type=f32))
    tq = ch[...] + q0[...] + q1[...]
    ecp = jnp.sum(tq * tq)
    dip = jnp.sum(pos[...] * tq, axis=0, keepdims=True)
    lane = jax.lax.broadcasted_iota(jnp.int32, (1, 128), 1)
    row = (jnp.where(lane == 0, e0p, 0.0) + jnp.where(lane == 1, ecp, 0.0)
           + jnp.where(lane == 2, dip[0, 0], 0.0)
           + jnp.where(lane == 3, dip[0, 1], 0.0)
           + jnp.where(lane == 4, dip[0, 2], 0.0))
    accv[...] += row

    @pl.when(i == pl.num_programs(0) - 1)
    def _():
        a = accv[...]
        dip2 = a[0, 2] ** 2 + a[0, 3] ** 2 + a[0, 4] ** 2
        val = (er0[0, 0] + er1[0, 0] + a[0, 0] + 0.5 * a[0, 1] + 1e-6 * dip2)
        out[...] = jnp.full((1, 1), val, jnp.float32)


def _t3(node_attrs, we0, charges, q0, q1, positions, er0, er1):
    f32 = jnp.float32
    nb = N // 5
    return pl.pallas_call(
        _t3_body,
        grid=(5,),
        in_specs=[pl.BlockSpec((nb, Z), lambda i: (i, 0)),
                  pl.BlockSpec((Z, 1), lambda i: (0, 0)),
                  pl.BlockSpec((nb, 1), lambda i: (i, 0)),
                  pl.BlockSpec((nb, 1), lambda i: (i, 0)),
                  pl.BlockSpec((nb, 1), lambda i: (i, 0)),
                  pl.BlockSpec((nb, 3), lambda i: (i, 0)),
                  pl.BlockSpec((1, 1), lambda i: (0, 0)),
                  pl.BlockSpec((1, 1), lambda i: (0, 0))],
        out_specs=pl.BlockSpec((1, 1), lambda i: (0, 0)),
        out_shape=jax.ShapeDtypeStruct((1, 1), f32),
        scratch_shapes=[pltpu.VMEM((1, 128), f32)],
    )(node_attrs, we0, charges, q0, q1, positions, er0, er1)


def kernel(positions, node_attrs, charges, shifts, W_E0, W_embed, W_msg, W1, W2, W3, W4, W_prod, W_read, W_q, edge_index, batch, ptr):
    n_nodes = positions.shape[0]
    sender = edge_index[0]
    recv = edge_index[1]

    px = positions[:, 0]
    py = positions[:, 1]
    pz = positions[:, 2]
    padidx = (jnp.arange(E, E_PAD, dtype=jnp.int32) % N).astype(jnp.int32)
    es_pad = jnp.concatenate([sender.astype(jnp.int32), padidx])
    er_pad = jnp.concatenate([recv.astype(jnp.int32), padidx])
    ersc = jnp.concatenate([
        recv.astype(jnp.int32),
        N + (jnp.arange(E, E_PAD, dtype=jnp.int32) % 16),
    ])
    dx, dy, dz = _edge_vectors(px, py, pz, es_pad, er_pad)

    ef8, sh3 = _t1a(dx.reshape(ER, 128), dy.reshape(ER, 128),
                    dz.reshape(ER, 128))
    ef2 = ef8.reshape(NB, E_PAD)
    tpw0 = _t1b(ef2, W1[0].T, W2[0].T, W3[0].T, W4[0])
    tpw1 = _t1b(ef2, W1[1].T, W2[1].T, W3[1].T, W4[1])
    sh_s = sh3.reshape(3, E_PAD)
    sh_b = jnp.concatenate(
        [jnp.ones((1, E_PAD), jnp.float32), sh_s], axis=0)
    sh_b = jnp.broadcast_to(sh_b[:, :, None], (4, E_PAD, 16))
    zrows = jnp.zeros((RPT, F), jnp.float32)

    pre = _t0(node_attrs, W_embed @ W_msg[0])

    acc4 = _aggregate(pre, tpw0, sh_b, es_pad, ersc, zrows)
    pre1, er0, q0 = _t2(acc4, W_prod[0], W_read[0][:, None], W_q[0][:, None],
                        W_msg[1], True)
    acc4 = _aggregate(pre1, tpw1, sh_b, es_pad, ersc, zrows)
    _, er1, q1 = _t2(acc4, W_prod[1], W_read[1][:, None], W_q[1][:, None],
                     W_msg[1], False)

    out = _t3(node_attrs, W_E0[:, None], charges[:, None],
              q0[:N], q1[:N], positions, er0, er1)
    return out[0]
